# Initial kernel scaffold; baseline (speedup 1.0000x reference)
import jax
import jax.numpy as jnp
from jax.experimental import pallas as pl


def kernel(logits, k):
    raise NotImplementedError("write your pallas kernel here")


# speedup vs baseline: 59.9225x; 59.9225x over previous
import functools

import jax
import jax.numpy as jnp
from jax.experimental import pallas as pl
from jax.experimental.pallas import tpu as pltpu

BLANK = 0
EXPAND_BEAM = 2.3
NEG_INF = -1e9
K = 4
_SENTINEL = -1e30
_INT_MAX = 2**31 - 1


def _fused_kernel(x_ref, vals_ref, idx_ref, m_ref, s_ref, tv_ref, ti_ref,
                  *, n_cols, nb):
    j = pl.program_id(1)
    R, C = x_ref.shape
    x = x_ref[...]
    col_ids = j * C + jax.lax.broadcasted_iota(jnp.int32, (R, C), 1)
    if n_cols % C != 0:
        x = jnp.where(col_ids < n_cols, x, _SENTINEL)

    @pl.when(j == 0)
    def _init():
        m_ref[...] = jnp.full((R, 1), _SENTINEL, jnp.float32)
        s_ref[...] = jnp.zeros((R, 1), jnp.float32)
        tv_ref[...] = jnp.full((R, K), _SENTINEL, jnp.float32)
        ti_ref[...] = jnp.full((R, K), _INT_MAX, jnp.int32)

    bm = jnp.max(x, axis=1, keepdims=True)
    m_old = m_ref[...]
    m_new = jnp.maximum(m_old, bm)
    s_ref[...] = s_ref[...] * jnp.exp(m_old - m_new) + jnp.sum(
        jnp.exp(x - m_new), axis=1, keepdims=True)
    m_ref[...] = m_new

    xb = x
    bv, bi = [], []
    for t in range(K):
        mv = bm if t == 0 else jnp.max(xb, axis=1, keepdims=True)
        mi = jnp.min(jnp.where(xb == mv, col_ids, _INT_MAX), axis=1,
                     keepdims=True)
        bv.append(mv)
        bi.append(mi)
        if t < K - 1:
            xb = jnp.where(col_ids == mi, _SENTINEL, xb)

    cv = jnp.concatenate([tv_ref[...]] + bv, axis=1)
    ci = jnp.concatenate([ti_ref[...]] + bi, axis=1)
    nv, ni = [], []
    for t in range(K):
        mv = jnp.max(cv, axis=1, keepdims=True)
        mi = jnp.min(jnp.where(cv == mv, ci, _INT_MAX), axis=1, keepdims=True)
        nv.append(mv)
        ni.append(mi)
        chosen = (cv == mv) & (ci == mi)
        cv = jnp.where(chosen, _SENTINEL, cv)
        ci = jnp.where(chosen, _INT_MAX, ci)
    tv = jnp.concatenate(nv, axis=1)
    ti = jnp.concatenate(ni, axis=1)
    tv_ref[...] = tv
    ti_ref[...] = ti

    @pl.when(j == nb - 1)
    def _finish():
        lse = m_ref[...] + jnp.log(s_ref[...])
        vv = tv - lse
        is_blank = ti[:, 0:1] == BLANK
        best = jnp.where(is_blank, vv[:, 1:2], vv[:, 0:1])
        keep = vv >= best - EXPAND_BEAM
        vals_ref[...] = jnp.where(keep, vv, NEG_INF)
        idx_ref[...] = ti


@functools.partial(jax.jit, static_argnames=("rows_blk", "cols_blk"))
def _run(logits, rows_blk, cols_blk):
    n_rows, n_cols = logits.shape
    nb = pl.cdiv(n_cols, cols_blk)
    grid = (n_rows // rows_blk, nb)
    out = pl.pallas_call(
        functools.partial(_fused_kernel, n_cols=n_cols, nb=nb),
        grid=grid,
        in_specs=[pl.BlockSpec((rows_blk, cols_blk), lambda i, j: (i, j))],
        out_specs=[
            pl.BlockSpec((rows_blk, K), lambda i, j: (i, 0)),
            pl.BlockSpec((rows_blk, K), lambda i, j: (i, 0)),
        ],
        out_shape=[
            jax.ShapeDtypeStruct((n_rows, K), jnp.float32),
            jax.ShapeDtypeStruct((n_rows, K), jnp.int32),
        ],
        scratch_shapes=[
            pltpu.VMEM((rows_blk, 1), jnp.float32),
            pltpu.VMEM((rows_blk, 1), jnp.float32),
            pltpu.VMEM((rows_blk, K), jnp.float32),
            pltpu.VMEM((rows_blk, K), jnp.int32),
        ],
    )(logits)
    return out[0], out[1]


def kernel(logits, k):
    del k
    n_rows, _ = logits.shape
    rows_blk = 256 if n_rows % 256 == 0 else n_rows
    return _run(logits, rows_blk, 2048)
